# baseline (speedup 1.0000x reference)
import functools

import jax
import jax.numpy as jnp
from jax import lax
from jax.experimental import pallas as pl
from jax.experimental.pallas import tpu as pltpu
from jax.experimental.pallas import tpu_sc as plsc


def _sc_broadcast_rows(pe2d, batch, length):
    d = pe2d.shape[1]
    info = plsc.get_sparse_core_info()
    nc, ns = info.num_cores, info.num_subcores
    nw = nc * ns
    slab = 32
    rows_per_w = -(-length // nw)
    rows_per_w = -(-rows_per_w // slab) * slab
    nslab = rows_per_w // slab
    assert length >= rows_per_w

    mesh = plsc.VectorSubcoreMesh(core_axis_name="c", subcore_axis_name="s")

    nchunk = nw // batch
    chunk = -(-length // nchunk)

    @functools.partial(
        pl.kernel,
        mesh=mesh,
        out_type=jax.ShapeDtypeStruct((batch, length, d), jnp.float32),
        scratch_types=[pltpu.SemaphoreType.DMA],
        compiler_params=pltpu.CompilerParams(use_tc_tiling_on_sc=False),
    )
    def k(pe_hbm, out_hbm, sem):
        wid = lax.axis_index("s") * nc + lax.axis_index("c")
        b = wid // nchunk
        ci = wid % nchunk
        base = jnp.minimum(ci * chunk, length - chunk)
        pltpu.make_async_copy(
            pe_hbm.at[pl.ds(base, chunk), :],
            out_hbm.at[b, pl.ds(base, chunk), :],
            sem,
        ).start()
        pltpu.make_async_copy(
            pe_hbm.at[pl.ds(base, chunk), :],
            out_hbm.at[b, pl.ds(base, chunk), :],
            sem,
        ).wait()

    return k(pe2d)


def kernel(x, pe):
    b, s, _ = x.shape
    length = 2 * s - 1
    return _sc_broadcast_rows(pe[0], b, length)


# speedup vs baseline: 5.6989x; 5.6989x over previous
import functools

import jax
import jax.numpy as jnp
from jax import lax
from jax.experimental import pallas as pl
from jax.experimental.pallas import tpu as pltpu
from jax.experimental.pallas import tpu_sc as plsc


def _sc_broadcast_rows(pe2d, batch, length):
    d = pe2d.shape[1]
    info = plsc.get_sparse_core_info()
    nc, ns = info.num_cores, info.num_subcores
    nw = nc * ns
    slab = 32
    rows_per_w = -(-length // nw)
    rows_per_w = -(-rows_per_w // slab) * slab
    nslab = rows_per_w // slab
    assert length >= rows_per_w

    mesh = plsc.ScalarSubcoreMesh(axis_name="c")
    chunk = 512
    half = -(-length // (2 * chunk))
    nbuf = 3

    @functools.partial(
        pl.kernel,
        mesh=mesh,
        out_type=jax.ShapeDtypeStruct((batch, length, d), jnp.float32),
        scratch_types=[
            [pltpu.VMEM_SHARED((chunk, d), jnp.float32) for _ in range(nbuf)],
            [pltpu.SemaphoreType.DMA for _ in range(nbuf)],
            [pltpu.SemaphoreType.DMA for _ in range(nbuf)],
        ],
        compiler_params=pltpu.CompilerParams(use_tc_tiling_on_sc=False),
    )
    def k(pe_hbm, out_hbm, bufs, gsems, wsems):
        sid = lax.axis_index("c")

        def base(i):
            return jnp.minimum((sid * half + i) * chunk, length - chunk)

        def gcopy(i):
            return pltpu.make_async_copy(
                pe_hbm.at[pl.ds(base(i), chunk), :],
                bufs[i % nbuf], gsems[i % nbuf])

        def wcopy(i, b):
            return pltpu.make_async_copy(
                bufs[i % nbuf],
                out_hbm.at[b, pl.ds(base(i), chunk), :],
                wsems[i % nbuf])

        gcopy(0).start()
        if half > 1:
            gcopy(1).start()
        for i in range(half):
            gcopy(i).wait()
            for b in range(batch):
                wcopy(i, b).start()
            if i > 0:
                for b in range(batch):
                    wcopy(i - 1, b).wait()
            if i + 2 < half:
                gcopy(i + 2).start()
        for b in range(batch):
            wcopy(half - 1, b).wait()

    return k(pe2d)


def kernel(x, pe):
    b, s, _ = x.shape
    length = 2 * s - 1
    return _sc_broadcast_rows(pe[0], b, length)


# speedup vs baseline: 24.6494x; 4.3253x over previous
import functools

import jax
import jax.numpy as jnp
from jax.experimental import pallas as pl
from jax.experimental.pallas import tpu as pltpu


def _tc_broadcast_rows(pe2d, batch, length):
    d = pe2d.shape[1]
    bs = 512
    grid = -(-length // bs)

    def body(pe_ref, out_ref):
        out_ref[...] = jnp.broadcast_to(pe_ref[...][None], (batch, bs, d))

    return pl.pallas_call(
        body,
        grid=(grid,),
        in_specs=[pl.BlockSpec((bs, d), lambda i: (i, 0))],
        out_specs=pl.BlockSpec((batch, bs, d), lambda i: (0, i, 0)),
        out_shape=jax.ShapeDtypeStruct((batch, length, d), jnp.float32),
        compiler_params=pltpu.CompilerParams(
            dimension_semantics=("arbitrary",),
        ),
    )(pe2d)


def kernel(x, pe):
    b, s, _ = x.shape
    length = 2 * s - 1
    return _tc_broadcast_rows(pe[0], b, length)
